# Initial kernel scaffold; baseline (speedup 1.0000x reference)
import jax
import jax.numpy as jnp
from jax.experimental import pallas as pl


def kernel(batch, corrupted_batch, entity_emb, relation_emb):
    raise NotImplementedError("write your pallas kernel here")


# speedup vs baseline: 1.1083x; 1.1083x over previous
import functools

import jax
import jax.numpy as jnp
from jax import lax
from jax.experimental import pallas as pl
from jax.experimental.pallas import tpu as pltpu
from jax.experimental.pallas import tpu_sc as plsc

LANES = 16
CHUNK = 128

_info = plsc.get_sparse_core_info()
_NC, _NS = _info.num_cores, _info.num_subcores
_NW = _NC * _NS


def _shuffle(x, idx):
    dnums = lax.GatherDimensionNumbers(
        offset_dims=(), collapsed_slice_dims=(0,), start_index_map=(0,))
    return lax.gather(x, idx[:, None], dnums, slice_sizes=(1,),
                      mode=lax.GatherScatterMode.PROMISE_IN_BOUNDS)


def _allsum(x):
    for k in (8, 4, 2, 1):
        x = x + _shuffle(x, lax.iota(jnp.int32, LANES) ^ k)
    return x


def _vrsqrt(x):
    xi = lax.bitcast_convert_type(x, jnp.int32)
    yi = jnp.int32(0x5F3759DF) - (xi >> 1)
    y = lax.bitcast_convert_type(yi, jnp.float32)
    xh = x * jnp.float32(-0.5)
    for _ in range(2):
        y = y * (jnp.float32(1.5) + xh * y * y)
    return y


@functools.lru_cache(maxsize=None)
def _make(batch_n, dim):
    assert dim % LANES == 0
    vpr = dim // LANES
    bpw = batch_n // _NW
    assert batch_n % (_NW * CHUNK) == 0
    chunks = bpw // CHUNK

    @functools.partial(
        pl.kernel,
        out_type=(
            jax.ShapeDtypeStruct((batch_n, dim), jnp.float32),
            jax.ShapeDtypeStruct((batch_n, dim), jnp.float32),
        ),
        mesh=plsc.VectorSubcoreMesh(core_axis_name="c", subcore_axis_name="s"),
        compiler_params=pltpu.CompilerParams(use_tc_tiling_on_sc=False),
        scratch_types=[
            pltpu.VMEM((CHUNK,), jnp.int32),
            pltpu.VMEM((CHUNK,), jnp.int32),
            pltpu.VMEM((CHUNK,), jnp.int32),
            pltpu.VMEM((CHUNK, dim), jnp.float32),
            pltpu.VMEM((CHUNK, dim), jnp.float32),
            pltpu.VMEM((CHUNK, dim), jnp.float32),
            pltpu.VMEM((CHUNK, dim), jnp.float32),
            pltpu.SemaphoreType.DMA,
        ],
    )
    def transe_sc(idx_hbm, ent_hbm, rel_hbm, out0, out1,
                  hi_v, ti_v, ri_v, h_v, t_v, r_v, o_v, sem):
        wid = lax.axis_index("s") * _NC + lax.axis_index("c")
        for p, out_hbm in ((0, out0), (1, out1)):
            for c in range(chunks):
                gbase = pl.multiple_of(wid * bpw + c * CHUNK, CHUNK)
                pltpu.sync_copy(idx_hbm.at[3 * p + 0, pl.ds(gbase, CHUNK)], hi_v)
                pltpu.sync_copy(idx_hbm.at[3 * p + 1, pl.ds(gbase, CHUNK)], ti_v)
                pltpu.sync_copy(idx_hbm.at[3 * p + 2, pl.ds(gbase, CHUNK)], ri_v)
                ch = pltpu.async_copy(ent_hbm.at[hi_v], h_v, sem)
                ct = pltpu.async_copy(ent_hbm.at[ti_v], t_v, sem)
                cr = pltpu.async_copy(rel_hbm.at[ri_v], r_v, sem)
                ch.wait()
                ct.wait()
                cr.wait()

                def row(i, carry):
                    hv = [h_v[i, pl.ds(LANES * j, LANES)] for j in range(vpr)]
                    tv = [t_v[i, pl.ds(LANES * j, LANES)] for j in range(vpr)]
                    rv = [r_v[i, pl.ds(LANES * j, LANES)] for j in range(vpr)]
                    hss = hv[0] * hv[0]
                    tss = tv[0] * tv[0]
                    for j in range(1, vpr):
                        hss += hv[j] * hv[j]
                        tss += tv[j] * tv[j]
                    a = _vrsqrt(jnp.maximum(_allsum(hss), jnp.float32(1e-24)))
                    b = _vrsqrt(jnp.maximum(_allsum(tss), jnp.float32(1e-24)))
                    for j in range(vpr):
                        o_v[i, pl.ds(LANES * j, LANES)] = a * hv[j] + (rv[j] - b * tv[j])
                    return carry

                lax.fori_loop(0, CHUNK, row, 0)
                pltpu.sync_copy(o_v, out_hbm.at[pl.ds(gbase, CHUNK)])

    return transe_sc


def kernel(batch, corrupted_batch, entity_emb, relation_emb):
    idx = jnp.concatenate([batch, corrupted_batch], axis=0).astype(jnp.int32)
    out0, out1 = _make(batch.shape[1], entity_emb.shape[1])(
        idx, entity_emb, relation_emb)
    return (out0, out1)


# speedup vs baseline: 1.1390x; 1.0277x over previous
import functools

import jax
import jax.numpy as jnp
from jax import lax
from jax.experimental import pallas as pl
from jax.experimental.pallas import tpu as pltpu
from jax.experimental.pallas import tpu_sc as plsc

LANES = 16

_info = plsc.get_sparse_core_info()
_NC, _NS = _info.num_cores, _info.num_subcores
_NW = _NC * _NS


def _shuffle(x, idx):
    dnums = lax.GatherDimensionNumbers(
        offset_dims=(), collapsed_slice_dims=(0,), start_index_map=(0,))
    return lax.gather(x, idx[:, None], dnums, slice_sizes=(1,),
                      mode=lax.GatherScatterMode.PROMISE_IN_BOUNDS)


def _allsum(x):
    for k in (8, 4, 2, 1):
        x = x + _shuffle(x, lax.iota(jnp.int32, LANES) ^ k)
    return x


def _vrsqrt(x):
    xi = lax.bitcast_convert_type(x, jnp.int32)
    yi = jnp.int32(0x5F3759DF) - (xi >> 1)
    y = lax.bitcast_convert_type(yi, jnp.float32)
    xh = x * jnp.float32(-0.5)
    for _ in range(2):
        y = y * (jnp.float32(1.5) + xh * y * y)
    return y


@functools.lru_cache(maxsize=None)
def _make(batch_n, dim):
    assert dim % LANES == 0
    vpr = dim // LANES
    assert batch_n % (_NW * 8) == 0
    bpw = batch_n // _NW

    @functools.partial(
        pl.kernel,
        out_type=(
            jax.ShapeDtypeStruct((batch_n, dim), jnp.float32),
            jax.ShapeDtypeStruct((batch_n, dim), jnp.float32),
        ),
        mesh=plsc.VectorSubcoreMesh(core_axis_name="c", subcore_axis_name="s"),
        compiler_params=pltpu.CompilerParams(use_tc_tiling_on_sc=False),
        scratch_types=[
            pltpu.VMEM((2 * bpw,), jnp.int32),
            pltpu.VMEM((2 * bpw,), jnp.int32),
            pltpu.VMEM((bpw,), jnp.int32),
            pltpu.VMEM((bpw,), jnp.int32),
            pltpu.VMEM((2 * bpw, dim), jnp.float32),
            pltpu.VMEM((bpw, dim), jnp.float32),
            pltpu.SemaphoreType.DMA,
            pltpu.SemaphoreType.DMA,
        ],
    )
    def transe_sc(idx_hbm, ent_hbm, rel_hbm, out0, out1,
                  e0_v, e1_v, r0_v, r1_v, ht_v, r_v, sem_g, sem_w):
        wid = lax.axis_index("s") * _NC + lax.axis_index("c")
        wbase = pl.multiple_of(wid * bpw, bpw)
        ci = [pltpu.async_copy(idx_hbm.at[wid, pl.ds(k * bpw, sz * bpw)], dst,
                               sem_g)
              for k, sz, dst in ((0, 2, e0_v), (2, 2, e1_v),
                                 (4, 1, r0_v), (5, 1, r1_v))]
        for c in ci:
            c.wait()

        def compute_pass():
            def row(i, carry):
                hv = [ht_v[i, pl.ds(LANES * j, LANES)] for j in range(vpr)]
                tv = [ht_v[bpw + i, pl.ds(LANES * j, LANES)] for j in range(vpr)]
                rv = [r_v[i, pl.ds(LANES * j, LANES)] for j in range(vpr)]
                hss = hv[0] * hv[0]
                tss = tv[0] * tv[0]
                for j in range(1, vpr):
                    hss += hv[j] * hv[j]
                    tss += tv[j] * tv[j]
                a = _vrsqrt(jnp.maximum(_allsum(hss), jnp.float32(1e-24)))
                b = _vrsqrt(jnp.maximum(_allsum(tss), jnp.float32(1e-24)))
                for j in range(vpr):
                    r_v[i, pl.ds(LANES * j, LANES)] = (
                        a * hv[j] + (rv[j] - b * tv[j]))
                return carry

            lax.fori_loop(0, bpw, row, 0)

        ge0 = pltpu.async_copy(ent_hbm.at[e0_v], ht_v, sem_g)
        gr0 = pltpu.async_copy(rel_hbm.at[r0_v], r_v, sem_g)
        ge0.wait()
        gr0.wait()
        compute_pass()
        wb0 = pltpu.async_copy(r_v, out0.at[pl.ds(wbase, bpw)], sem_w)
        ge1 = pltpu.async_copy(ent_hbm.at[e1_v], ht_v, sem_g)
        wb0.wait()
        gr1 = pltpu.async_copy(rel_hbm.at[r1_v], r_v, sem_g)
        ge1.wait()
        gr1.wait()
        compute_pass()
        pltpu.sync_copy(r_v, out1.at[pl.ds(wbase, bpw)])

    return transe_sc


def kernel(batch, corrupted_batch, entity_emb, relation_emb):
    nw, b = _NW, batch.shape[1]
    bpw = b // nw
    rows = [batch[0], batch[1], corrupted_batch[0], corrupted_batch[1],
            batch[2], corrupted_batch[2]]
    idx = jnp.concatenate(
        [r.astype(jnp.int32).reshape(nw, bpw) for r in rows], axis=1)
    out0, out1 = _make(b, entity_emb.shape[1])(idx, entity_emb, relation_emb)
    return (out0, out1)


# speedup vs baseline: 1.6983x; 1.4910x over previous
import functools

import jax
import jax.numpy as jnp
from jax import lax
from jax.experimental import pallas as pl
from jax.experimental.pallas import tpu as pltpu
from jax.experimental.pallas import tpu_sc as plsc

LANES = 16

_info = plsc.get_sparse_core_info()
_NC, _NS = _info.num_cores, _info.num_subcores
_NW = _NC * _NS


def _shuffle(x, idx):
    dnums = lax.GatherDimensionNumbers(
        offset_dims=(), collapsed_slice_dims=(0,), start_index_map=(0,))
    return lax.gather(x, idx[:, None], dnums, slice_sizes=(1,),
                      mode=lax.GatherScatterMode.PROMISE_IN_BOUNDS)


def _allsum(x):
    for k in (8, 4, 2, 1):
        x = x + _shuffle(x, lax.iota(jnp.int32, LANES) ^ k)
    return x


def _vrsqrt(x):
    xi = lax.bitcast_convert_type(x, jnp.int32)
    yi = jnp.int32(0x5F3759DF) - (xi >> 1)
    y = lax.bitcast_convert_type(yi, jnp.float32)
    xh = x * jnp.float32(-0.5)
    for _ in range(2):
        y = y * (jnp.float32(1.5) + xh * y * y)
    return y


@functools.lru_cache(maxsize=None)
def _make(batch_n, dim):
    assert dim % LANES == 0 and dim <= 128
    vpr = dim // LANES
    bpw = batch_n // _NW
    C = 64
    spp = bpw // C
    S = 2 * spp
    NB = 3
    ipw = 2 * 3 * bpw

    @functools.partial(
        pl.kernel,
        out_type=(
            jax.ShapeDtypeStruct((batch_n, dim), jnp.float32),
            jax.ShapeDtypeStruct((batch_n, dim), jnp.float32),
        ),
        mesh=plsc.VectorSubcoreMesh(core_axis_name="c", subcore_axis_name="s"),
        compiler_params=pltpu.CompilerParams(use_tc_tiling_on_sc=True),
        scratch_types=[
            pltpu.VMEM((ipw,), jnp.int32),
            pltpu.VMEM((NB * C, dim), jnp.float32),
            pltpu.VMEM((NB * C, dim), jnp.float32),
            pltpu.VMEM((NB * C, dim), jnp.float32),
            pltpu.SemaphoreType.DMA,
            pltpu.SemaphoreType.DMA,
            pltpu.SemaphoreType.DMA,
            pltpu.SemaphoreType.DMA,
            pltpu.SemaphoreType.DMA,
            pltpu.SemaphoreType.DMA,
        ],
    )
    def transe_sc(idx_hbm, ent_hbm, rel_hbm, out0, out1,
                  idx_v, hb, tb, rb, g0, g1, g2, w0, w1, w2):
        wid = lax.axis_index("s") * _NC + lax.axis_index("c")
        pltpu.sync_copy(idx_hbm.at[pl.ds(wid * ipw, ipw)], idx_v)

        outs = (out0, out1)
        gsems = (g0, g1, g2)
        wsems = (w0, w1, w2)

        def issue_gather(s):
            p, k = s // spp, s % spp
            boff = (s % NB) * C
            hbase = p * 3 * bpw + k * C
            tbase = hbase + bpw
            rbase = hbase + 2 * bpw
            sem = gsems[s % NB]

            def blk(bi, carry):
                hids = idx_v[pl.ds(hbase + bi * LANES, LANES)]
                tids = idx_v[pl.ds(tbase + bi * LANES, LANES)]
                rids = idx_v[pl.ds(rbase + bi * LANES, LANES)]
                row0 = boff + bi * LANES
                for j in range(LANES):
                    pltpu.async_copy(ent_hbm.at[hids[j]], hb.at[row0 + j], sem)
                    pltpu.async_copy(ent_hbm.at[tids[j]], tb.at[row0 + j], sem)
                    pltpu.async_copy(rel_hbm.at[rids[j]], rb.at[row0 + j], sem)
                return carry

            lax.fori_loop(0, C // LANES, blk, 0)

        def wait_gather(s):
            sem = gsems[s % NB]

            def one(i, carry):
                pltpu.make_async_copy(ent_hbm.at[0], hb.at[0], sem).wait()
                return carry

            lax.fori_loop(0, 3 * C, one, 0)

        def compute(s):
            boff = (s % NB) * C

            def row(i, carry):
                hv = [hb[boff + i, pl.ds(LANES * j, LANES)] for j in range(vpr)]
                tv = [tb[boff + i, pl.ds(LANES * j, LANES)] for j in range(vpr)]
                rv = [rb[boff + i, pl.ds(LANES * j, LANES)] for j in range(vpr)]
                hss = hv[0] * hv[0]
                tss = tv[0] * tv[0]
                for j in range(1, vpr):
                    hss += hv[j] * hv[j]
                    tss += tv[j] * tv[j]
                a = _vrsqrt(jnp.maximum(_allsum(hss), jnp.float32(1e-24)))
                c = _vrsqrt(jnp.maximum(_allsum(tss), jnp.float32(1e-24)))
                for j in range(vpr):
                    hb[boff + i, pl.ds(LANES * j, LANES)] = (
                        a * hv[j] + (rv[j] - c * tv[j]))
                return carry

            lax.fori_loop(0, C, row, 0)

        def issue_writeback(s):
            p, k = s // spp, s % spp
            boff = (s % NB) * C
            return pltpu.async_copy(
                hb.at[pl.ds(boff, C)],
                outs[p].at[pl.ds(wid * bpw + k * C, C)],
                wsems[s % NB])

        issue_gather(0)
        issue_gather(1)
        wbs = {}
        for s in range(S):
            wait_gather(s)
            compute(s)
            wbs[s] = issue_writeback(s)
            if s >= 1:
                wbs[s - 1].wait()
            if s + 2 < S:
                issue_gather(s + 2)
        wbs[S - 1].wait()

    return transe_sc


def kernel(batch, corrupted_batch, entity_emb, relation_emb):
    nw, b = _NW, batch.shape[1]
    bpw = b // nw
    dim = entity_emb.shape[1]

    def streams(h, t, r):
        return jnp.stack([h.astype(jnp.int32).reshape(nw, bpw),
                          t.astype(jnp.int32).reshape(nw, bpw),
                          r.astype(jnp.int32).reshape(nw, bpw)], axis=1)

    s0 = streams(batch[0], batch[1], batch[2])
    s1 = streams(corrupted_batch[0], corrupted_batch[1],
                 corrupted_batch[2])
    idx = jnp.concatenate([s0, s1], axis=1).reshape(-1)

    out0, out1 = _make(b, dim)(idx, entity_emb, relation_emb)
    return (out0, out1)
